# baseline (speedup 1.0000x reference)
import functools

import jax
import jax.numpy as jnp
import numpy as np
from jax.experimental import pallas as pl
from jax.experimental.pallas import tpu as pltpu

_STRIDES = [4, 8, 16, 32, 64]
_SIZES = [(128, 128), (64, 64), (32, 32), (16, 16), (8, 8)]
_B = 2
_C = 256
_NA = 3


def _head_kernel(x_ref, w9_ref, cb_ref, wh_ref, bh_ref, o_ref):
    r = pl.program_id(1)
    rb = o_ref.shape[1]
    w = o_ref.shape[2]
    wp = w + 2
    acc = None
    for dy in range(3):
        a = x_ref[0, pl.ds(r * rb + dy, rb), :, :]
        a2 = a.reshape(rb * wp, _C)
        for dx in range(3):
            y = jax.lax.dot_general(
                a2, w9_ref[dy * 3 + dx],
                (((1,), (0,)), ((), ())),
                precision=jax.lax.Precision.DEFAULT,
                preferred_element_type=jnp.float32,
            ).reshape(rb, wp, _C)
            t = y[:, dx:dx + w, :]
            acc = t if acc is None else acc + t
    z = jnp.maximum(acc + cb_ref[0, :], 0.0)
    h = jax.lax.dot_general(
        z.reshape(rb * w, _C), wh_ref[...],
        (((1,), (0,)), ((), ())),
        precision=jax.lax.Precision.DEFAULT,
        preferred_element_type=jnp.float32,
    ) + bh_ref[0, :]
    o_ref[0] = h.reshape(rb, w, 24)


def _run_level_head(xp, w9, cb, wh, bh, h, w):
    rb = 8
    grid = (_B, h // rb)
    return pl.pallas_call(
        _head_kernel,
        grid=grid,
        in_specs=[
            pl.BlockSpec((1, h + 2, w + 2, _C), lambda b, r: (b, 0, 0, 0)),
            pl.BlockSpec((9, _C, _C), lambda b, r: (0, 0, 0)),
            pl.BlockSpec((1, _C), lambda b, r: (0, 0)),
            pl.BlockSpec((_C, 24), lambda b, r: (0, 0)),
            pl.BlockSpec((1, 24), lambda b, r: (0, 0)),
        ],
        out_specs=pl.BlockSpec((1, rb, w, 24), lambda b, r: (b, r, 0, 0)),
        out_shape=jax.ShapeDtypeStruct((_B, h, w, 24), jnp.float32),
        compiler_params=pltpu.CompilerParams(vmem_limit_bytes=110 * 1024 * 1024),
    )(xp, w9, cb, wh, bh)


def _make_anchors(h, w, stride):
    base = stride * 8.0
    ratios = jnp.array([0.5, 1.0, 2.0], jnp.float32)
    aw = base / jnp.sqrt(ratios)
    ah = base * jnp.sqrt(ratios)
    xs = (jnp.arange(w, dtype=jnp.float32) + 0.5) * stride
    ys = (jnp.arange(h, dtype=jnp.float32) + 0.5) * stride
    cy, cx = jnp.meshgrid(ys, xs, indexing='ij')
    cx = jnp.broadcast_to(cx[None], (_NA, h, w))
    cy = jnp.broadcast_to(cy[None], (_NA, h, w))
    aw = jnp.broadcast_to(aw[:, None, None], (_NA, h, w))
    ah = jnp.broadcast_to(ah[:, None, None], (_NA, h, w))
    return jnp.stack([cx, cy, aw, ah], -1).reshape(_NA * h * w, 4)


def _decode_midpoint(reg, anchors):
    cx, cy, aw, ah = anchors[:, 0], anchors[:, 1], anchors[:, 2], anchors[:, 3]
    dx, dy, dw, dh, da, db = [reg[..., i] for i in range(6)]
    x = dx * aw + cx
    y = dy * ah + cy
    w = aw * jnp.exp(jnp.clip(dw, -8.0, 8.0))
    h = ah * jnp.exp(jnp.clip(dh, -8.0, 8.0))
    da_ = da * w
    db_ = db * h
    v1 = jnp.stack([x + da_, y - h / 2], -1)
    v2 = jnp.stack([x + w / 2, y + db_], -1)
    v3 = jnp.stack([x - da_, y + h / 2], -1)
    v4 = jnp.stack([x - w / 2, y - db_], -1)
    return jnp.stack([v1, v2, v3, v4], -2)


def _nms_keep(boxes, scores, thr):
    n = boxes.shape[0]
    b = boxes
    area = jnp.clip(b[:, 2] - b[:, 0], 0) * jnp.clip(b[:, 3] - b[:, 1], 0)
    x1 = jnp.maximum(b[:, None, 0], b[None, :, 0])
    y1 = jnp.maximum(b[:, None, 1], b[None, :, 1])
    x2 = jnp.minimum(b[:, None, 2], b[None, :, 2])
    y2 = jnp.minimum(b[:, None, 3], b[None, :, 3])
    inter = jnp.clip(x2 - x1, 0) * jnp.clip(y2 - y1, 0)
    iou = inter / (area[:, None] + area[None, :] - inter + 1e-9)

    def body(i, keep):
        sup = (iou[i] > thr) & (jnp.arange(n) > i) & keep[i]
        return keep & (~sup)

    return jax.lax.fori_loop(0, n, body, jnp.ones((n,), bool))


def kernel(p2, p3, p4, p5, p6, conv_w, conv_b, reg_w, reg_b, obj_w, obj_b):
    feats = [p2, p3, p4, p5, p6]
    w9 = jnp.transpose(conv_w, (2, 3, 1, 0)).reshape(9, _C, _C)
    cb = conv_b.reshape(1, _C)
    wh = jnp.concatenate([
        reg_w.reshape(18, _C).T,
        obj_w.reshape(3, _C).T,
        jnp.zeros((_C, 3), jnp.float32),
    ], axis=1)
    bh = jnp.concatenate([reg_b, obj_b, jnp.zeros((3,), jnp.float32)]).reshape(1, 24)

    level_props = []
    level_scores = []
    level_keeps = []
    for s_idx, v in enumerate(feats):
        b, _, h, w = v.shape
        xp = jnp.pad(jnp.transpose(v, (0, 2, 3, 1)),
                     ((0, 0), (1, 1), (1, 1), (0, 0)))
        head = _run_level_head(xp, w9, cb, wh, bh, h, w)
        reg = head[..., :18].reshape(b, h, w, _NA, 6)
        reg = jnp.transpose(reg, (0, 3, 1, 2, 4)).reshape(b, _NA * h * w, 6)
        obj = jnp.transpose(head[..., 18:21], (0, 3, 1, 2)).reshape(b, _NA * h * w)
        anchors = _make_anchors(h, w, _STRIDES[s_idx])
        props_b = []
        scores_b = []
        keeps_b = []
        for bi in range(b):
            k = min(2000, obj.shape[1])
            vals, idx = jax.lax.top_k(obj[bi], k)
            reg_k = reg[bi][idx]
            verts_k = _decode_midpoint(reg_k, anchors[idx])
            hbb_k = jnp.concatenate([verts_k.min(axis=1), verts_k.max(axis=1)], -1)
            keep = _nms_keep(hbb_k, vals, 0.8)
            props_b.append(verts_k)
            scores_b.append(vals)
            keeps_b.append(keep)
        level_props.append(props_b)
        level_scores.append(scores_b)
        level_keeps.append(keeps_b)

    out_p = jnp.zeros((_B, 1000, 8), jnp.float32)
    out_s = jnp.zeros((_B, 1000), jnp.float32)
    for bi in range(_B):
        merged = jnp.concatenate([level_scores[l][bi] for l in range(len(feats))])
        kept = jnp.concatenate([level_keeps[l][bi] for l in range(len(feats))])
        props = jnp.concatenate([level_props[l][bi] for l in range(len(feats))], 0)
        props = props.reshape(-1, 8)
        kk = min(1000, merged.shape[0])
        masked = jnp.where(kept, merged, -jnp.inf)
        _, tidx = jax.lax.top_k(masked, kk)
        topmask = jnp.zeros((merged.shape[0],), bool).at[tidx].set(True)
        sel = topmask & kept
        dst = jnp.where(sel, jnp.cumsum(sel) - 1, 1000)
        out_p = out_p.at[bi, dst].set(props, mode='drop')
        out_s = out_s.at[bi, dst].set(merged, mode='drop')
    return out_p, out_s


# speedup vs baseline: 10.7345x; 10.7345x over previous
import functools

import jax
import jax.numpy as jnp
import numpy as np
from jax.experimental import pallas as pl
from jax.experimental.pallas import tpu as pltpu

_STRIDES = [4, 8, 16, 32, 64]
_SIZES = [(128, 128), (64, 64), (32, 32), (16, 16), (8, 8)]
_B = 2
_C = 256
_NA = 3


def _head_kernel(x_ref, w9_ref, cb_ref, wh_ref, bh_ref, o_ref):
    r = pl.program_id(1)
    rb = o_ref.shape[1]
    w = o_ref.shape[2]
    wp = w + 2
    acc = None
    for dy in range(3):
        a = x_ref[0, pl.ds(r * rb + dy, rb), :, :]
        a2 = a.reshape(rb * wp, _C)
        for dx in range(3):
            y = jax.lax.dot_general(
                a2, w9_ref[dy * 3 + dx],
                (((1,), (0,)), ((), ())),
                precision=jax.lax.Precision.DEFAULT,
                preferred_element_type=jnp.float32,
            ).reshape(rb, wp, _C)
            t = y[:, dx:dx + w, :]
            acc = t if acc is None else acc + t
    z = jnp.maximum(acc + cb_ref[0, :], 0.0)
    h = jax.lax.dot_general(
        z.reshape(rb * w, _C), wh_ref[...],
        (((1,), (0,)), ((), ())),
        precision=jax.lax.Precision.DEFAULT,
        preferred_element_type=jnp.float32,
    ) + bh_ref[0, :]
    o_ref[0] = h.reshape(rb, w, 24)


def _run_level_head(xp, w9, cb, wh, bh, h, w):
    rb = 8
    grid = (_B, h // rb)
    return pl.pallas_call(
        _head_kernel,
        grid=grid,
        in_specs=[
            pl.BlockSpec((1, h + 2, w + 2, _C), lambda b, r: (b, 0, 0, 0)),
            pl.BlockSpec((9, _C, _C), lambda b, r: (0, 0, 0)),
            pl.BlockSpec((1, _C), lambda b, r: (0, 0)),
            pl.BlockSpec((_C, 24), lambda b, r: (0, 0)),
            pl.BlockSpec((1, 24), lambda b, r: (0, 0)),
        ],
        out_specs=pl.BlockSpec((1, rb, w, 24), lambda b, r: (b, r, 0, 0)),
        out_shape=jax.ShapeDtypeStruct((_B, h, w, 24), jnp.float32),
        compiler_params=pltpu.CompilerParams(vmem_limit_bytes=110 * 1024 * 1024),
    )(xp, w9, cb, wh, bh)


def _make_anchors(h, w, stride):
    base = stride * 8.0
    ratios = jnp.array([0.5, 1.0, 2.0], jnp.float32)
    aw = base / jnp.sqrt(ratios)
    ah = base * jnp.sqrt(ratios)
    xs = (jnp.arange(w, dtype=jnp.float32) + 0.5) * stride
    ys = (jnp.arange(h, dtype=jnp.float32) + 0.5) * stride
    cy, cx = jnp.meshgrid(ys, xs, indexing='ij')
    cx = jnp.broadcast_to(cx[None], (_NA, h, w))
    cy = jnp.broadcast_to(cy[None], (_NA, h, w))
    aw = jnp.broadcast_to(aw[:, None, None], (_NA, h, w))
    ah = jnp.broadcast_to(ah[:, None, None], (_NA, h, w))
    return jnp.stack([cx, cy, aw, ah], -1).reshape(_NA * h * w, 4)


def _decode_midpoint(reg, anchors):
    cx, cy, aw, ah = anchors[:, 0], anchors[:, 1], anchors[:, 2], anchors[:, 3]
    dx, dy, dw, dh, da, db = [reg[..., i] for i in range(6)]
    x = dx * aw + cx
    y = dy * ah + cy
    w = aw * jnp.exp(jnp.clip(dw, -8.0, 8.0))
    h = ah * jnp.exp(jnp.clip(dh, -8.0, 8.0))
    da_ = da * w
    db_ = db * h
    v1 = jnp.stack([x + da_, y - h / 2], -1)
    v2 = jnp.stack([x + w / 2, y + db_], -1)
    v3 = jnp.stack([x - da_, y + h / 2], -1)
    v4 = jnp.stack([x - w / 2, y - db_], -1)
    return jnp.stack([v1, v2, v3, v4], -2)


_NMS_N = 2048
_NMS_K = 256


def _nms_kernel(bc_ref, br_ref, keep_ref, iou_scr, keep_scr):
    thr = 0.8
    n = _NMS_N
    kk = _NMS_K
    x1c = bc_ref[0, 0:1, :]
    y1c = bc_ref[0, 1:2, :]
    x2c = bc_ref[0, 2:3, :]
    y2c = bc_ref[0, 3:4, :]
    areac = jnp.maximum(x2c - x1c, 0.0) * jnp.maximum(y2c - y1c, 0.0)
    keep_scr[0:1, :] = jnp.ones((1, n), jnp.float32)
    iotan = jax.lax.broadcasted_iota(jnp.int32, (1, n), 1)
    iotak = jax.lax.broadcasted_iota(jnp.int32, (1, kk), 1)
    for blk in range(n // kk):
        base = blk * kk
        x1r = br_ref[0, pl.ds(base, kk), 0:1]
        y1r = br_ref[0, pl.ds(base, kk), 1:2]
        x2r = br_ref[0, pl.ds(base, kk), 2:3]
        y2r = br_ref[0, pl.ds(base, kk), 3:4]
        arear = jnp.maximum(x2r - x1r, 0.0) * jnp.maximum(y2r - y1r, 0.0)
        xx1 = jnp.maximum(x1r, x1c)
        yy1 = jnp.maximum(y1r, y1c)
        xx2 = jnp.minimum(x2r, x2c)
        yy2 = jnp.minimum(y2r, y2c)
        inter = jnp.maximum(xx2 - xx1, 0.0) * jnp.maximum(yy2 - yy1, 0.0)
        iou_scr[...] = inter / (arear + areac - inter + 1e-9)

        def body(i, carry):
            row = iou_scr[pl.ds(i, 1), pl.ds(base, kk)]
            kb = keep_scr[0:1, pl.ds(base, kk)]
            ki = jnp.sum(jnp.where(iotak == i, kb, 0.0))
            supf = jnp.where((row > thr) & (iotak > i), ki, 0.0)
            keep_scr[0:1, pl.ds(base, kk)] = kb * (1.0 - supf)
            return carry

        jax.lax.fori_loop(0, kk, body, 0)
        kbcol = keep_scr[0:1, pl.ds(base, kk)].reshape(kk, 1)
        masked = iou_scr[...] * kbcol
        colmax = jnp.max(masked, axis=0, keepdims=True)
        supv = jnp.where((colmax > thr) & (iotan >= base + kk), 1.0, 0.0)
        keep_scr[0:1, :] = keep_scr[0:1, :] * (1.0 - supv)
    keep_ref[...] = keep_scr[0:1, :].reshape(1, 1, _NMS_N)


def _run_nms(boxes_r):
    from jax.experimental.pallas import tpu as _pltpu
    q = boxes_r.shape[0]
    boxes_c = jnp.transpose(boxes_r, (0, 2, 1))
    return pl.pallas_call(
        _nms_kernel,
        grid=(q,),
        in_specs=[
            pl.BlockSpec((1, 4, _NMS_N), lambda i: (i, 0, 0)),
            pl.BlockSpec((1, _NMS_N, 4), lambda i: (i, 0, 0)),
        ],
        out_specs=pl.BlockSpec((1, 1, _NMS_N), lambda i: (i, 0, 0)),
        out_shape=jax.ShapeDtypeStruct((q, 1, _NMS_N), jnp.float32),
        scratch_shapes=[
            _pltpu.VMEM((_NMS_K, _NMS_N), jnp.float32),
            _pltpu.VMEM((1, _NMS_N), jnp.float32),
        ],
        compiler_params=pltpu.CompilerParams(
            vmem_limit_bytes=110 * 1024 * 1024),
    )(boxes_c, boxes_r)


def kernel(p2, p3, p4, p5, p6, conv_w, conv_b, reg_w, reg_b, obj_w, obj_b):
    feats = [p2, p3, p4, p5, p6]
    w9 = jnp.transpose(conv_w, (2, 3, 1, 0)).reshape(9, _C, _C)
    cb = conv_b.reshape(1, _C)
    wh = jnp.concatenate([
        reg_w.reshape(18, _C).T,
        obj_w.reshape(3, _C).T,
        jnp.zeros((_C, 3), jnp.float32),
    ], axis=1)
    bh = jnp.concatenate([reg_b, obj_b, jnp.zeros((3,), jnp.float32)]).reshape(1, 24)

    level_props = []
    level_scores = []
    level_keeps = []
    for s_idx, v in enumerate(feats):
        b, _, h, w = v.shape
        xp = jnp.pad(jnp.transpose(v, (0, 2, 3, 1)),
                     ((0, 0), (1, 1), (1, 1), (0, 0)))
        head = _run_level_head(xp, w9, cb, wh, bh, h, w)
        reg = head[..., :18].reshape(b, h, w, _NA, 6)
        reg = jnp.transpose(reg, (0, 3, 1, 2, 4)).reshape(b, _NA * h * w, 6)
        obj = jnp.transpose(head[..., 18:21], (0, 3, 1, 2)).reshape(b, _NA * h * w)
        anchors = _make_anchors(h, w, _STRIDES[s_idx])
        props_b = []
        scores_b = []
        hbb_b = []
        for bi in range(b):
            k = min(2000, obj.shape[1])
            vals, idx = jax.lax.top_k(obj[bi], k)
            reg_k = reg[bi][idx]
            verts_k = _decode_midpoint(reg_k, anchors[idx])
            hbb_k = jnp.concatenate([verts_k.min(axis=1), verts_k.max(axis=1)], -1)
            props_b.append(verts_k)
            scores_b.append(vals)
            hbb_b.append(jnp.pad(hbb_k, ((0, _NMS_N - k), (0, 0))))
        level_props.append(props_b)
        level_scores.append(scores_b)
        level_keeps.append(hbb_b)

    boxes_r = jnp.stack([level_keeps[l][bi]
                         for l in range(len(feats)) for bi in range(_B)])
    keepm = _run_nms(boxes_r)
    for l in range(len(feats)):
        for bi in range(_B):
            k = level_scores[l][bi].shape[0]
            level_keeps[l][bi] = keepm[l * _B + bi, 0, :k].astype(bool)

    out_p = jnp.zeros((_B, 1000, 8), jnp.float32)
    out_s = jnp.zeros((_B, 1000), jnp.float32)
    for bi in range(_B):
        merged = jnp.concatenate([level_scores[l][bi] for l in range(len(feats))])
        kept = jnp.concatenate([level_keeps[l][bi] for l in range(len(feats))])
        props = jnp.concatenate([level_props[l][bi] for l in range(len(feats))], 0)
        props = props.reshape(-1, 8)
        kk = min(1000, merged.shape[0])
        masked = jnp.where(kept, merged, -jnp.inf)
        _, tidx = jax.lax.top_k(masked, kk)
        topmask = jnp.zeros((merged.shape[0],), bool).at[tidx].set(True)
        sel = topmask & kept
        dst = jnp.where(sel, jnp.cumsum(sel) - 1, 1000)
        out_p = out_p.at[bi, dst].set(props, mode='drop')
        out_s = out_s.at[bi, dst].set(merged, mode='drop')
    return out_p, out_s


# speedup vs baseline: 38.2850x; 3.5665x over previous
import functools

import jax
import jax.numpy as jnp
import numpy as np
from jax.experimental import pallas as pl
from jax.experimental.pallas import tpu as pltpu

_STRIDES = [4, 8, 16, 32, 64]
_SIZES = [(128, 128), (64, 64), (32, 32), (16, 16), (8, 8)]
_B = 2
_C = 256
_NA = 3


def _head_kernel(x_ref, w9_ref, cb_ref, wh_ref, bh_ref, o_ref):
    r = pl.program_id(1)
    rb = o_ref.shape[1]
    w = o_ref.shape[2]
    wp = w + 2
    acc = None
    for dy in range(3):
        a = x_ref[0, pl.ds(r * rb + dy, rb), :, :]
        a2 = a.reshape(rb * wp, _C)
        for dx in range(3):
            y = jax.lax.dot_general(
                a2, w9_ref[dy * 3 + dx],
                (((1,), (0,)), ((), ())),
                precision=jax.lax.Precision.DEFAULT,
                preferred_element_type=jnp.float32,
            ).reshape(rb, wp, _C)
            t = y[:, dx:dx + w, :]
            acc = t if acc is None else acc + t
    z = jnp.maximum(acc + cb_ref[0, :], 0.0)
    h = jax.lax.dot_general(
        z.reshape(rb * w, _C), wh_ref[...],
        (((1,), (0,)), ((), ())),
        precision=jax.lax.Precision.DEFAULT,
        preferred_element_type=jnp.float32,
    ) + bh_ref[0, :]
    o_ref[0] = h.reshape(rb, w, 24)


def _run_level_head(xp, w9, cb, wh, bh, h, w):
    rb = 8
    grid = (_B, h // rb)
    return pl.pallas_call(
        _head_kernel,
        grid=grid,
        in_specs=[
            pl.BlockSpec((1, h + 2, w + 2, _C), lambda b, r: (b, 0, 0, 0)),
            pl.BlockSpec((9, _C, _C), lambda b, r: (0, 0, 0)),
            pl.BlockSpec((1, _C), lambda b, r: (0, 0)),
            pl.BlockSpec((_C, 24), lambda b, r: (0, 0)),
            pl.BlockSpec((1, 24), lambda b, r: (0, 0)),
        ],
        out_specs=pl.BlockSpec((1, rb, w, 24), lambda b, r: (b, r, 0, 0)),
        out_shape=jax.ShapeDtypeStruct((_B, h, w, 24), jnp.float32),
        compiler_params=pltpu.CompilerParams(vmem_limit_bytes=110 * 1024 * 1024),
    )(xp, w9, cb, wh, bh)


def _make_anchors(h, w, stride):
    base = stride * 8.0
    ratios = jnp.array([0.5, 1.0, 2.0], jnp.float32)
    aw = base / jnp.sqrt(ratios)
    ah = base * jnp.sqrt(ratios)
    xs = (jnp.arange(w, dtype=jnp.float32) + 0.5) * stride
    ys = (jnp.arange(h, dtype=jnp.float32) + 0.5) * stride
    cy, cx = jnp.meshgrid(ys, xs, indexing='ij')
    cx = jnp.broadcast_to(cx[None], (_NA, h, w))
    cy = jnp.broadcast_to(cy[None], (_NA, h, w))
    aw = jnp.broadcast_to(aw[:, None, None], (_NA, h, w))
    ah = jnp.broadcast_to(ah[:, None, None], (_NA, h, w))
    return jnp.stack([cx, cy, aw, ah], -1).reshape(_NA * h * w, 4)


def _decode_midpoint(reg, anchors):
    cx, cy, aw, ah = anchors[:, 0], anchors[:, 1], anchors[:, 2], anchors[:, 3]
    dx, dy, dw, dh, da, db = [reg[..., i] for i in range(6)]
    x = dx * aw + cx
    y = dy * ah + cy
    w = aw * jnp.exp(jnp.clip(dw, -8.0, 8.0))
    h = ah * jnp.exp(jnp.clip(dh, -8.0, 8.0))
    da_ = da * w
    db_ = db * h
    v1 = jnp.stack([x + da_, y - h / 2], -1)
    v2 = jnp.stack([x + w / 2, y + db_], -1)
    v3 = jnp.stack([x - da_, y + h / 2], -1)
    v4 = jnp.stack([x - w / 2, y - db_], -1)
    return jnp.stack([v1, v2, v3, v4], -2)


_NMS_N = 2048
_NMS_K = 256


def _nms_kernel(bc_ref, br_ref, keep_ref, sup_scr):
    thr = 0.8
    n = _NMS_N
    kk = _NMS_K
    x1c = bc_ref[0, 0:1, :]
    y1c = bc_ref[0, 1:2, :]
    x2c = bc_ref[0, 2:3, :]
    y2c = bc_ref[0, 3:4, :]
    areac = jnp.maximum(x2c - x1c, 0.0) * jnp.maximum(y2c - y1c, 0.0)
    iotan = jax.lax.broadcasted_iota(jnp.int32, (1, n), 1)
    for blk in range(n // kk):
        base = blk * kk
        x1r = br_ref[0, pl.ds(base, kk), 0:1]
        y1r = br_ref[0, pl.ds(base, kk), 1:2]
        x2r = br_ref[0, pl.ds(base, kk), 2:3]
        y2r = br_ref[0, pl.ds(base, kk), 3:4]
        arear = jnp.maximum(x2r - x1r, 0.0) * jnp.maximum(y2r - y1r, 0.0)
        xx1 = jnp.maximum(x1r, x1c)
        yy1 = jnp.maximum(y1r, y1c)
        xx2 = jnp.minimum(x2r, x2c)
        yy2 = jnp.minimum(y2r, y2c)
        inter = jnp.maximum(xx2 - xx1, 0.0) * jnp.maximum(yy2 - yy1, 0.0)
        iou = inter / (arear + areac - inter + 1e-9)
        rowidx = base + jax.lax.broadcasted_iota(jnp.int32, (kk, 1), 0)
        sup_scr[pl.ds(base, kk), :] = jnp.where(
            (iou > thr) & (iotan > rowidx), 1.0, 0.0)

    def body(carry):
        keep, _ = carry
        s = jax.lax.dot_general(
            keep, sup_scr[...], (((1,), (0,)), ((), ())),
            precision=jax.lax.Precision.DEFAULT,
            preferred_element_type=jnp.float32)
        keep_new = jnp.where(s > 0.5, 0.0, 1.0)
        changed = jnp.sum(jnp.abs(keep_new - keep)) > 0.0
        return keep_new, changed

    def cond(carry):
        return carry[1]

    keep0 = jnp.ones((8, n), jnp.float32)
    keep, _ = jax.lax.while_loop(cond, body, (keep0, True))
    keep_ref[...] = keep[0:1, :].reshape(1, 1, _NMS_N)


def _run_nms(boxes_r):
    from jax.experimental.pallas import tpu as _pltpu
    q = boxes_r.shape[0]
    boxes_c = jnp.transpose(boxes_r, (0, 2, 1))
    return pl.pallas_call(
        _nms_kernel,
        grid=(q,),
        in_specs=[
            pl.BlockSpec((1, 4, _NMS_N), lambda i: (i, 0, 0)),
            pl.BlockSpec((1, _NMS_N, 4), lambda i: (i, 0, 0)),
        ],
        out_specs=pl.BlockSpec((1, 1, _NMS_N), lambda i: (i, 0, 0)),
        out_shape=jax.ShapeDtypeStruct((q, 1, _NMS_N), jnp.float32),
        scratch_shapes=[
            _pltpu.VMEM((_NMS_N, _NMS_N), jnp.float32),
        ],
        compiler_params=pltpu.CompilerParams(
            vmem_limit_bytes=110 * 1024 * 1024),
    )(boxes_c, boxes_r)


def kernel(p2, p3, p4, p5, p6, conv_w, conv_b, reg_w, reg_b, obj_w, obj_b):
    feats = [p2, p3, p4, p5, p6]
    w9 = jnp.transpose(conv_w, (2, 3, 1, 0)).reshape(9, _C, _C)
    cb = conv_b.reshape(1, _C)
    wh = jnp.concatenate([
        reg_w.reshape(18, _C).T,
        obj_w.reshape(3, _C).T,
        jnp.zeros((_C, 3), jnp.float32),
    ], axis=1)
    bh = jnp.concatenate([reg_b, obj_b, jnp.zeros((3,), jnp.float32)]).reshape(1, 24)

    level_props = []
    level_scores = []
    level_keeps = []
    for s_idx, v in enumerate(feats):
        b, _, h, w = v.shape
        xp = jnp.pad(jnp.transpose(v, (0, 2, 3, 1)),
                     ((0, 0), (1, 1), (1, 1), (0, 0)))
        head = _run_level_head(xp, w9, cb, wh, bh, h, w)
        reg = head[..., :18].reshape(b, h, w, _NA, 6)
        reg = jnp.transpose(reg, (0, 3, 1, 2, 4)).reshape(b, _NA * h * w, 6)
        obj = jnp.transpose(head[..., 18:21], (0, 3, 1, 2)).reshape(b, _NA * h * w)
        anchors = _make_anchors(h, w, _STRIDES[s_idx])
        props_b = []
        scores_b = []
        hbb_b = []
        for bi in range(b):
            k = min(2000, obj.shape[1])
            vals, idx = jax.lax.top_k(obj[bi], k)
            reg_k = reg[bi][idx]
            verts_k = _decode_midpoint(reg_k, anchors[idx])
            hbb_k = jnp.concatenate([verts_k.min(axis=1), verts_k.max(axis=1)], -1)
            props_b.append(verts_k)
            scores_b.append(vals)
            hbb_b.append(jnp.pad(hbb_k, ((0, _NMS_N - k), (0, 0))))
        level_props.append(props_b)
        level_scores.append(scores_b)
        level_keeps.append(hbb_b)

    boxes_r = jnp.stack([level_keeps[l][bi]
                         for l in range(len(feats)) for bi in range(_B)])
    keepm = _run_nms(boxes_r)
    for l in range(len(feats)):
        for bi in range(_B):
            k = level_scores[l][bi].shape[0]
            level_keeps[l][bi] = keepm[l * _B + bi, 0, :k].astype(bool)

    out_p = jnp.zeros((_B, 1000, 8), jnp.float32)
    out_s = jnp.zeros((_B, 1000), jnp.float32)
    for bi in range(_B):
        merged = jnp.concatenate([level_scores[l][bi] for l in range(len(feats))])
        kept = jnp.concatenate([level_keeps[l][bi] for l in range(len(feats))])
        props = jnp.concatenate([level_props[l][bi] for l in range(len(feats))], 0)
        props = props.reshape(-1, 8)
        kk = min(1000, merged.shape[0])
        masked = jnp.where(kept, merged, -jnp.inf)
        _, tidx = jax.lax.top_k(masked, kk)
        topmask = jnp.zeros((merged.shape[0],), bool).at[tidx].set(True)
        sel = topmask & kept
        dst = jnp.where(sel, jnp.cumsum(sel) - 1, 1000)
        out_p = out_p.at[bi, dst].set(props, mode='drop')
        out_s = out_s.at[bi, dst].set(merged, mode='drop')
    return out_p, out_s
